# baseline (speedup 1.0000x reference)
import functools
import math

import jax
import jax.numpy as jnp
from jax import lax
from jax.experimental import pallas as pl
from jax.experimental.pallas import tpu as pltpu
from jax.experimental.pallas import tpu_sc as plsc

_NUM_BUCKETS = 32
_MAX_DISTANCE = 128
_HEADS = 16
_Q = 2048
_K = 2048
_NB = 128
_TAB = 4096
_PAD = 4224


def _shift_table_kernel(bias_ref, out_ref):
    r = lax.broadcasted_iota(jnp.int32, (8, _PAD), 0)
    j = lax.broadcasted_iota(jnp.int32, (8, _PAD), 1)
    relative_position = j + (7 - r) - (_Q - 1)

    num_buckets = _NUM_BUCKETS // 2
    relative_buckets = (relative_position > 0).astype(jnp.int32) * num_buckets
    n = jnp.abs(relative_position)
    max_exact = num_buckets // 2
    is_small = n < max_exact
    nf = n.astype(jnp.float32)
    rp_if_large = max_exact + jnp.log(nf / max_exact) / math.log(
        _MAX_DISTANCE / max_exact
    ) * (num_buckets - max_exact)
    rp_if_large = jnp.minimum(
        rp_if_large, jnp.full_like(rp_if_large, num_buckets - 1)
    )
    buckets_f = relative_buckets.astype(jnp.float32) + jnp.where(
        is_small, nf, rp_if_large
    )
    bucket = buckets_f.astype(jnp.int32)

    srev0 = jnp.zeros((8, _PAD), jnp.float32)
    for b in range(_NUM_BUCKETS):
        srev0 = jnp.where(bucket == b, bias_ref[0, 0, b], srev0)

    for m in range(_NB // 8):
        start = 120 - 8 * m
        out_ref[0, 8 * m : 8 * m + 8, :] = srev0[:, start : start + _TAB]


def _build_shift_tables(relative_attention_bias):
    bias_t = relative_attention_bias.T.reshape(_HEADS, 1, _NUM_BUCKETS)
    return pl.pallas_call(
        _shift_table_kernel,
        grid=(_HEADS,),
        in_specs=[pl.BlockSpec((1, 1, _NUM_BUCKETS), lambda h: (h, 0, 0))],
        out_specs=pl.BlockSpec((1, _NB, _TAB), lambda h: (h, 0, 0)),
        out_shape=jax.ShapeDtypeStruct((_HEADS, _NB, _TAB), jnp.float32),
    )(bias_t)


_RT = _Q // 8
_CT = _K // 128
_TILES_PER_WORKER = (_RT // 2) * _CT
_CHUNK = 16


def _expand_kernel(s128_hbm, out_hbm, sem):
    c = lax.axis_index("c")
    s = lax.axis_index("s")
    h = s
    rbase = c * (_RT // 2)

    def issue(i):
        rt = rbase + lax.div(i, _CT)
        ct = lax.rem(i, _CT)
        b0 = 8 * lax.rem(rt, 16)
        joff = 128 * (15 - lax.div(rt, 16) + ct)
        pltpu.make_async_copy(
            s128_hbm.at[h, pl.ds(b0, 8), pl.ds(joff, 128)],
            out_hbm.at[0, h, pl.ds(8 * rt, 8), pl.ds(128 * ct, 128)],
            sem,
        ).start()

    def drain_one():
        pltpu.make_async_copy(
            s128_hbm.at[h, pl.ds(0, 8), pl.ds(0, 128)],
            out_hbm.at[0, h, pl.ds(0, 8), pl.ds(0, 128)],
            sem,
        ).wait()

    for jj in range(_CHUNK):
        issue(jj)

    def body(i, carry):
        base = i * _CHUNK
        for jj in range(_CHUNK):
            issue(base + jj)
        for jj in range(_CHUNK):
            drain_one()
        return carry

    lax.fori_loop(1, _TILES_PER_WORKER // _CHUNK, body, 0)
    for jj in range(_CHUNK):
        drain_one()


def kernel(encoder_hidden, decoder_hidden, relative_attention_bias):
    del encoder_hidden, decoder_hidden
    s128 = _build_shift_tables(relative_attention_bias)

    mesh = plsc.VectorSubcoreMesh(core_axis_name="c", subcore_axis_name="s")
    expand = functools.partial(
        pl.kernel,
        mesh=mesh,
        out_type=jax.ShapeDtypeStruct((1, _HEADS, _Q, _K), jnp.float32),
        scratch_types=[pltpu.SemaphoreType.DMA],
    )(_expand_kernel)
    return expand(s128)


# speedup vs baseline: 36.2886x; 36.2886x over previous
import functools
import math

import jax
import jax.numpy as jnp
from jax import lax
from jax.experimental import pallas as pl
from jax.experimental.pallas import tpu as pltpu
from jax.experimental.pallas import tpu_sc as plsc

_NUM_BUCKETS = 32
_MAX_DISTANCE = 128
_HEADS = 16
_Q = 2048
_K = 2048
_NB = 128
_TAB = 4096
_PAD = 4224


def _shift_table_kernel(bias_ref, out_ref):
    r = lax.broadcasted_iota(jnp.int32, (8, _PAD), 0)
    j = lax.broadcasted_iota(jnp.int32, (8, _PAD), 1)
    relative_position = j + (7 - r) - (_Q - 1)

    num_buckets = _NUM_BUCKETS // 2
    relative_buckets = (relative_position > 0).astype(jnp.int32) * num_buckets
    n = jnp.abs(relative_position)
    max_exact = num_buckets // 2
    is_small = n < max_exact
    nf = n.astype(jnp.float32)
    rp_if_large = max_exact + jnp.log(nf / max_exact) / math.log(
        _MAX_DISTANCE / max_exact
    ) * (num_buckets - max_exact)
    rp_if_large = jnp.minimum(
        rp_if_large, jnp.full_like(rp_if_large, num_buckets - 1)
    )
    buckets_f = relative_buckets.astype(jnp.float32) + jnp.where(
        is_small, nf, rp_if_large
    )
    bucket = buckets_f.astype(jnp.int32)

    srev0 = jnp.zeros((8, _PAD), jnp.float32)
    for b in range(_NUM_BUCKETS):
        srev0 = jnp.where(bucket == b, bias_ref[0, 0, b], srev0)

    for m in range(_NB // 8):
        start = 120 - 8 * m
        out_ref[0, 8 * m : 8 * m + 8, :] = srev0[:, start : start + _TAB]


def _build_shift_tables(relative_attention_bias):
    bias_t = relative_attention_bias.T.reshape(_HEADS, 1, _NUM_BUCKETS)
    return pl.pallas_call(
        _shift_table_kernel,
        grid=(_HEADS,),
        in_specs=[pl.BlockSpec((1, 1, _NUM_BUCKETS), lambda h: (h, 0, 0))],
        out_specs=pl.BlockSpec((1, _NB, _TAB), lambda h: (h, 0, 0)),
        out_shape=jax.ShapeDtypeStruct((_HEADS, _NB, _TAB), jnp.float32),
    )(bias_t)


_SLAB_COLS = 256
_SLABS = 64
_WRITES_PER_SLAB = 32


def _expand_kernel(s128_hbm, out_hbm, buf0, buf1, sem_load, sem_store):
    c = lax.axis_index("c")
    s = lax.axis_index("s")
    h = s

    def slab_coff(k):
        g = 8 * c + lax.div(k, 8)
        return 128 * (15 - g) + _SLAB_COLS * lax.rem(k, 8)

    def start_load(k, buf):
        pltpu.make_async_copy(
            s128_hbm.at[h, :, pl.ds(slab_coff(k), _SLAB_COLS)],
            buf,
            sem_load,
        ).start()

    def wait_load():
        pltpu.make_async_copy(
            s128_hbm.at[h, :, pl.ds(0, _SLAB_COLS)], buf0, sem_load
        ).wait()

    def issue_writes(k, buf):
        g = 8 * c + lax.div(k, 8)
        cc = lax.rem(k, 8)
        for u in range(16):
            for w in range(2):
                pltpu.make_async_copy(
                    buf.at[pl.ds(8 * u, 8), pl.ds(128 * w, 128)],
                    out_hbm.at[
                        0,
                        h,
                        pl.ds(8 * (16 * g + u), 8),
                        pl.ds(128 * (2 * cc + w), 128),
                    ],
                    sem_store,
                ).start()

    def drain_writes():
        for _ in range(_WRITES_PER_SLAB):
            pltpu.make_async_copy(
                buf0.at[pl.ds(0, 8), pl.ds(0, 128)],
                out_hbm.at[0, h, pl.ds(0, 8), pl.ds(0, 128)],
                sem_store,
            ).wait()

    start_load(0, buf0)

    def body(t, carry):
        k0 = 2 * t
        k1 = 2 * t + 1
        wait_load()

        @pl.when(t > 0)
        def _():
            drain_writes()

        start_load(k1, buf1)
        issue_writes(k0, buf0)
        wait_load()
        drain_writes()
        start_load(jnp.minimum(k1 + 1, _SLABS - 1), buf0)
        issue_writes(k1, buf1)
        return carry

    lax.fori_loop(0, _SLABS // 2, body, 0)
    wait_load()
    drain_writes()


def kernel(encoder_hidden, decoder_hidden, relative_attention_bias):
    del encoder_hidden, decoder_hidden
    s128 = _build_shift_tables(relative_attention_bias)

    mesh = plsc.VectorSubcoreMesh(core_axis_name="c", subcore_axis_name="s")
    expand = functools.partial(
        pl.kernel,
        mesh=mesh,
        out_type=jax.ShapeDtypeStruct((1, _HEADS, _Q, _K), jnp.float32),
        scratch_types=[
            pltpu.VMEM((_NB, _SLAB_COLS), jnp.float32),
            pltpu.VMEM((_NB, _SLAB_COLS), jnp.float32),
            pltpu.SemaphoreType.DMA,
            pltpu.SemaphoreType.DMA,
        ],
    )(_expand_kernel)
    return expand(s128)


# speedup vs baseline: 59.0158x; 1.6263x over previous
import functools
import math

import jax
import jax.numpy as jnp
from jax import lax
from jax.experimental import pallas as pl
from jax.experimental.pallas import tpu as pltpu
from jax.experimental.pallas import tpu_sc as plsc

_NUM_BUCKETS = 32
_MAX_DISTANCE = 128
_HEADS = 16
_Q = 2048
_K = 2048
_NB = 128
_TAB = 4608
_PAD = 4736


def _shift_table_kernel(bias_ref, out_ref):
    r = lax.broadcasted_iota(jnp.int32, (8, _PAD), 0)
    j = lax.broadcasted_iota(jnp.int32, (8, _PAD), 1)
    relative_position = j + (7 - r) - (_Q - 1)

    num_buckets = _NUM_BUCKETS // 2
    relative_buckets = (relative_position > 0).astype(jnp.int32) * num_buckets
    n = jnp.abs(relative_position)
    max_exact = num_buckets // 2
    is_small = n < max_exact
    nf = n.astype(jnp.float32)
    rp_if_large = max_exact + jnp.log(nf / max_exact) / math.log(
        _MAX_DISTANCE / max_exact
    ) * (num_buckets - max_exact)
    rp_if_large = jnp.minimum(
        rp_if_large, jnp.full_like(rp_if_large, num_buckets - 1)
    )
    buckets_f = relative_buckets.astype(jnp.float32) + jnp.where(
        is_small, nf, rp_if_large
    )
    bucket = buckets_f.astype(jnp.int32)

    srev0 = jnp.zeros((8, _PAD), jnp.float32)
    for b in range(_NUM_BUCKETS):
        srev0 = jnp.where(bucket == b, bias_ref[0, 0, b], srev0)

    for m in range(_NB // 8):
        start = 120 - 8 * m
        out_ref[0, 8 * m : 8 * m + 8, :] = srev0[:, start : start + _TAB]


def _build_shift_tables(relative_attention_bias):
    bias_t = relative_attention_bias.T.reshape(_HEADS, 1, _NUM_BUCKETS)
    return pl.pallas_call(
        _shift_table_kernel,
        grid=(_HEADS,),
        in_specs=[pl.BlockSpec((1, 1, _NUM_BUCKETS), lambda h: (h, 0, 0))],
        out_specs=pl.BlockSpec((1, _NB, _TAB), lambda h: (h, 0, 0)),
        out_shape=jax.ShapeDtypeStruct((_HEADS, _NB, _TAB), jnp.float32),
    )(bias_t)


_WINS = (0, 7, 14, 21, 28)
_WIN_CT = 7
_WIN_COLS = 128 * _WIN_CT
_BUF_ROWS = 64


def _win_crange(w0, g):
    c_lo = max(0, w0 + g - 15)
    c_hi = min(15, w0 + g - 9)
    return (c_lo, c_hi) if c_lo <= c_hi else None


def _expand_kernel(s128_hbm, out_hbm, buf0, buf1, sem_load, sem_store):
    c = lax.axis_index("c")
    s = lax.axis_index("s")
    h = s
    rowbase = pl.multiple_of(_BUF_ROWS * c, 8)
    bufs = (buf0, buf1)

    def start_load(i):
        pltpu.make_async_copy(
            s128_hbm.at[
                h, pl.ds(rowbase, _BUF_ROWS), pl.ds(128 * _WINS[i], _WIN_COLS)
            ],
            bufs[i % 2],
            sem_load,
        ).start()

    def wait_load():
        pltpu.make_async_copy(
            s128_hbm.at[h, pl.ds(0, _BUF_ROWS), pl.ds(0, _WIN_COLS)],
            buf0,
            sem_load,
        ).wait()

    def issue_win(i):
        w0 = _WINS[i]
        buf = bufs[i % 2]
        for g in range(16):
            cr = _win_crange(w0, g)
            if cr is None:
                continue
            c_lo, c_hi = cr
            n = c_hi - c_lo + 1
            boff = 128 * (15 - g + c_lo - w0)

            def one(ul, carry, g=g, c_lo=c_lo, n=n, boff=boff, buf=buf):
                row = pl.multiple_of(128 * g + _BUF_ROWS * c + 8 * ul, 8)
                pltpu.make_async_copy(
                    buf.at[pl.ds(8 * ul, 8), pl.ds(boff, 128 * n)],
                    out_hbm.at[0, h, pl.ds(row, 8), pl.ds(128 * c_lo, 128 * n)],
                    sem_store,
                ).start()
                return carry

            lax.fori_loop(0, 8, one, 0)

    def drain_win(i):
        w0 = _WINS[i]
        for g in range(16):
            cr = _win_crange(w0, g)
            if cr is None:
                continue
            c_lo, c_hi = cr
            n = c_hi - c_lo + 1

            def one(ul, carry, n=n):
                pltpu.make_async_copy(
                    buf0.at[pl.ds(0, 8), pl.ds(0, 128 * n)],
                    out_hbm.at[0, h, pl.ds(0, 8), pl.ds(0, 128 * n)],
                    sem_store,
                ).wait()
                return carry

            lax.fori_loop(0, 8, one, 0)

    start_load(0)
    for i in range(len(_WINS)):
        wait_load()
        if i >= 1:
            drain_win(i - 1)
        if i + 1 < len(_WINS):
            start_load(i + 1)
        issue_win(i)
    drain_win(len(_WINS) - 1)


def kernel(encoder_hidden, decoder_hidden, relative_attention_bias):
    del encoder_hidden, decoder_hidden
    s128 = _build_shift_tables(relative_attention_bias)

    mesh = plsc.VectorSubcoreMesh(core_axis_name="c", subcore_axis_name="s")
    expand = functools.partial(
        pl.kernel,
        mesh=mesh,
        out_type=jax.ShapeDtypeStruct((1, _HEADS, _Q, _K), jnp.float32),
        scratch_types=[
            pltpu.VMEM((_BUF_ROWS, _WIN_COLS), jnp.float32),
            pltpu.VMEM((_BUF_ROWS, _WIN_COLS), jnp.float32),
            pltpu.SemaphoreType.DMA,
            pltpu.SemaphoreType.DMA,
        ],
    )(_expand_kernel)
    return expand(s128)


# speedup vs baseline: 60.9753x; 1.0332x over previous
import functools
import math

import jax
import jax.numpy as jnp
from jax import lax
from jax.experimental import pallas as pl
from jax.experimental.pallas import tpu as pltpu
from jax.experimental.pallas import tpu_sc as plsc

_NUM_BUCKETS = 32
_MAX_DISTANCE = 128
_HEADS = 16
_Q = 2048
_K = 2048
_NB = 128
_TAB = 3968
_PAD = 4096


def _shift_table_kernel(bias_ref, out_ref):
    r = lax.broadcasted_iota(jnp.int32, (8, _PAD), 0)
    j = lax.broadcasted_iota(jnp.int32, (8, _PAD), 1)
    relative_position = j + (7 - r) - (_Q - 1)

    num_buckets = _NUM_BUCKETS // 2
    relative_buckets = (relative_position > 0).astype(jnp.int32) * num_buckets
    n = jnp.abs(relative_position)
    max_exact = num_buckets // 2
    is_small = n < max_exact
    nf = n.astype(jnp.float32)
    rp_if_large = max_exact + jnp.log(nf / max_exact) / math.log(
        _MAX_DISTANCE / max_exact
    ) * (num_buckets - max_exact)
    rp_if_large = jnp.minimum(
        rp_if_large, jnp.full_like(rp_if_large, num_buckets - 1)
    )
    buckets_f = relative_buckets.astype(jnp.float32) + jnp.where(
        is_small, nf, rp_if_large
    )
    bucket = buckets_f.astype(jnp.int32)

    srev0 = jnp.zeros((8, _PAD), jnp.float32)
    for b in range(_NUM_BUCKETS):
        srev0 = jnp.where(bucket == b, bias_ref[0, 0, b], srev0)

    for m in range(_NB // 8):
        start = 120 - 8 * m
        out_ref[0, 8 * m : 8 * m + 8, :] = srev0[:, start : start + _TAB]


def _build_shift_tables(relative_attention_bias):
    bias_t = relative_attention_bias.T.reshape(_HEADS, 1, _NUM_BUCKETS)
    return pl.pallas_call(
        _shift_table_kernel,
        grid=(_HEADS,),
        in_specs=[pl.BlockSpec((1, 1, _NUM_BUCKETS), lambda h: (h, 0, 0))],
        out_specs=pl.BlockSpec((1, _NB, _TAB), lambda h: (h, 0, 0)),
        out_shape=jax.ShapeDtypeStruct((_HEADS, _NB, _TAB), jnp.float32),
    )(bias_t)


_WINS = (0, 7, 14, 21, 28)
_WIN_CT = 7
_WIN_COLS = 128 * _WIN_CT
_BUF_ROWS = 64
_LOAD_COLS = tuple(
    min(_WIN_COLS, _TAB - 128 * w0) for w0 in _WINS
)


def _win_crange(w0, g):
    c_lo = max(0, w0 + g - 15)
    c_hi = min(15, w0 + g - 9)
    return (c_lo, c_hi) if c_lo <= c_hi else None


def _expand_kernel(s128_hbm, out_hbm, buf0, buf1, sem_load, sem_store):
    c = lax.axis_index("c")
    s = lax.axis_index("s")
    h = s
    rowbase = pl.multiple_of(_BUF_ROWS * c, 8)
    bufs = (buf0, buf1)

    def start_load(i):
        pltpu.make_async_copy(
            s128_hbm.at[
                h, pl.ds(rowbase, _BUF_ROWS), pl.ds(128 * _WINS[i], _LOAD_COLS[i])
            ],
            bufs[i % 2].at[:, pl.ds(0, _LOAD_COLS[i])],
            sem_load,
        ).start()

    def wait_load(i):
        pltpu.make_async_copy(
            s128_hbm.at[h, pl.ds(0, _BUF_ROWS), pl.ds(0, _LOAD_COLS[i])],
            buf0.at[:, pl.ds(0, _LOAD_COLS[i])],
            sem_load,
        ).wait()

    def issue_win(i):
        w0 = _WINS[i]
        buf = bufs[i % 2]
        for g in range(16):
            cr = _win_crange(w0, g)
            if cr is None:
                continue
            c_lo, c_hi = cr
            n = c_hi - c_lo + 1
            boff = 128 * (15 - g + c_lo - w0)
            row = pl.multiple_of(128 * g + _BUF_ROWS * c, 8)
            pltpu.make_async_copy(
                buf.at[:, pl.ds(boff, 128 * n)],
                out_hbm.at[
                    0, h, pl.ds(row, _BUF_ROWS), pl.ds(128 * c_lo, 128 * n)
                ],
                sem_store,
            ).start()

    def drain_win(i):
        w0 = _WINS[i]
        for g in range(16):
            cr = _win_crange(w0, g)
            if cr is None:
                continue
            c_lo, c_hi = cr
            n = c_hi - c_lo + 1
            pltpu.make_async_copy(
                buf0.at[:, pl.ds(0, 128 * n)],
                out_hbm.at[0, h, pl.ds(0, _BUF_ROWS), pl.ds(0, 128 * n)],
                sem_store,
            ).wait()

    start_load(0)
    for i in range(len(_WINS)):
        wait_load(i)
        if i >= 1:
            drain_win(i - 1)
        if i + 1 < len(_WINS):
            start_load(i + 1)
        issue_win(i)
    drain_win(len(_WINS) - 1)


def kernel(encoder_hidden, decoder_hidden, relative_attention_bias):
    del encoder_hidden, decoder_hidden
    s128 = _build_shift_tables(relative_attention_bias)

    mesh = plsc.VectorSubcoreMesh(core_axis_name="c", subcore_axis_name="s")
    expand = functools.partial(
        pl.kernel,
        mesh=mesh,
        out_type=jax.ShapeDtypeStruct((1, _HEADS, _Q, _K), jnp.float32),
        scratch_types=[
            pltpu.VMEM((_BUF_ROWS, _WIN_COLS), jnp.float32),
            pltpu.VMEM((_BUF_ROWS, _WIN_COLS), jnp.float32),
            pltpu.SemaphoreType.DMA,
            pltpu.SemaphoreType.DMA,
        ],
    )(_expand_kernel)
    return expand(s128)
